# baseline (speedup 1.0000x reference)
import jax
import jax.numpy as jnp
from jax import lax
from jax.experimental import pallas as pl
from jax.experimental.pallas import tpu as pltpu
from jax.experimental.pallas import tpu_sc as plsc

D = 128

NC = 2
NS = 16
NW = NC * NS

CHUNK = 64
NPAD = 10240
ROWS_PER_TILE = NPAD // NS
CW = 8

SPLIT_A = 200
SPLIT_B = 114
MAXSPLIT = max(SPLIT_A, SPLIT_B)


def _sc_body(feats_hbm, src_hbm, dst_hbm, zrow_hbm, zcnt_hbm, ones_hbm,
             sums_out, cnts_out,
             sidx_v, didx_v, rows0_v, rows1_v, ones_v, cstage_v,
             acc_sh, cnt_sh,
             sem_g0, sem_g1, sem_s0, sem_s1, sem_c0, sem_c1):
  c = lax.axis_index("c")
  s = lax.axis_index("s")

  start = lax.select(c == 0, s * SPLIT_A, NS * SPLIT_A + s * SPLIT_B)
  n_half = lax.select(c == 0, SPLIT_A // 2, SPLIT_B // 2)

  row0 = s * ROWS_PER_TILE

  nz = ROWS_PER_TILE // CHUNK
  for k in range(nz):
    pltpu.async_copy(zrow_hbm, acc_sh.at[pl.ds(row0 + k * CHUNK, CHUNK)],
                     sem_s0)
    pltpu.async_copy(zcnt_hbm, cnt_sh.at[pl.ds(row0 + k * CHUNK, CHUNK)],
                     sem_s1)
  pltpu.async_copy(src_hbm.at[pl.ds(start, MAXSPLIT)], sidx_v, sem_g0)
  pltpu.async_copy(dst_hbm.at[pl.ds(start, MAXSPLIT)], didx_v, sem_g1)
  pltpu.async_copy(ones_hbm, ones_v, sem_c0)
  for k in range(nz):
    pltpu.make_async_copy(
        zrow_hbm, acc_sh.at[pl.ds(row0 + k * CHUNK, CHUNK)], sem_s0).wait()
    pltpu.make_async_copy(
        zcnt_hbm, cnt_sh.at[pl.ds(row0 + k * CHUNK, CHUNK)], sem_s1).wait()
  pltpu.make_async_copy(
      src_hbm.at[pl.ds(start, MAXSPLIT)], sidx_v, sem_g0).wait()
  pltpu.make_async_copy(
      dst_hbm.at[pl.ds(start, MAXSPLIT)], didx_v, sem_g1).wait()
  pltpu.make_async_copy(ones_hbm, ones_v, sem_c0).wait()
  plsc.subcore_barrier()

  H = CHUNK // 2

  def gather(j, rows_v, sem):
    pltpu.async_copy(feats_hbm.at[sidx_v.at[j, pl.ds(0, H)]],
                     rows_v.at[pl.ds(0, H)], sem)
    pltpu.async_copy(feats_hbm.at[sidx_v.at[j, pl.ds(H, H)]],
                     rows_v.at[pl.ds(H, H)], sem)

  def gather_wait(j, rows_v, sem):
    pltpu.make_async_copy(feats_hbm.at[sidx_v.at[j, pl.ds(0, H)]],
                          rows_v.at[pl.ds(0, H)], sem).wait()
    pltpu.make_async_copy(feats_hbm.at[sidx_v.at[j, pl.ds(H, H)]],
                          rows_v.at[pl.ds(H, H)], sem).wait()

  def scatter(j, rows_v, sem):
    return pltpu.async_copy(rows_v, acc_sh.at[didx_v.at[j]], sem, add=True)

  def counts(j, sem):
    return pltpu.async_copy(ones_v, cnt_sh.at[didx_v.at[j]], sem, add=True)

  gather(0, rows0_v, sem_g0)

  def body(i, carry):
    j0 = 2 * i
    j1 = j0 + 1
    gather_wait(j0, rows0_v, sem_g0)
    scatter(j0, rows0_v, sem_s0)

    @pl.when(i > 0)
    def _():
      pltpu.make_async_copy(rows1_v, acc_sh.at[didx_v.at[j1]], sem_s1).wait()
      pltpu.make_async_copy(ones_v, cnt_sh.at[didx_v.at[j1]], sem_c1).wait()

    counts(j0, sem_c0)
    gather(j1, rows1_v, sem_g1)

    gather_wait(j1, rows1_v, sem_g1)
    scatter(j1, rows1_v, sem_s1)
    pltpu.make_async_copy(rows0_v, acc_sh.at[didx_v.at[j0]], sem_s0).wait()
    pltpu.make_async_copy(ones_v, cnt_sh.at[didx_v.at[j0]], sem_c0).wait()
    counts(j1, sem_c1)

    @pl.when(i < n_half - 1)
    def _():
      gather(j0 + 2, rows0_v, sem_g0)

    return carry

  lax.fori_loop(0, n_half, body, 0)
  pltpu.make_async_copy(rows1_v, acc_sh.at[didx_v.at[0]], sem_s1).wait()
  pltpu.make_async_copy(ones_v, cnt_sh.at[didx_v.at[0]], sem_c1).wait()
  plsc.subcore_barrier()

  for k in range(nz):
    pltpu.async_copy(acc_sh.at[pl.ds(row0 + k * CHUNK, CHUNK)],
                     sums_out.at[c, pl.ds(row0 + k * CHUNK, CHUNK)], sem_s0)
    pltpu.async_copy(cnt_sh.at[pl.ds(row0 + k * CHUNK, CHUNK)],
                     cnts_out.at[c, pl.ds(row0 + k * CHUNK, CHUNK)], sem_s1)
  for k in range(nz):
    pltpu.make_async_copy(
        acc_sh.at[pl.ds(row0 + k * CHUNK, CHUNK)],
        sums_out.at[c, pl.ds(row0 + k * CHUNK, CHUNK)], sem_s0).wait()
    pltpu.make_async_copy(
        cnt_sh.at[pl.ds(row0 + k * CHUNK, CHUNK)],
        cnts_out.at[c, pl.ds(row0 + k * CHUNK, CHUNK)], sem_s1).wait()


def _tc_body(sums_ref, cnts_ref, w_ref, b_ref, out_ref):
  ssum = sums_ref[0] + sums_ref[1]
  cnt = cnts_ref[0, :, 0:1] + cnts_ref[1, :, 0:1]
  neigh = ssum / jnp.maximum(cnt, 1.0)
  acc = lax.dot_general(neigh, w_ref[...], (((1,), (1,)), ((), ())),
                        preferred_element_type=jnp.float32)
  out_ref[...] = jnp.maximum(acc + b_ref[...], 0.0)


def kernel(node_feats, edge_index, W, b):
  n, d = node_feats.shape
  e = edge_index.shape[1]
  src = edge_index[0].astype(jnp.int32)
  dst = edge_index[1].astype(jnp.int32)

  tot_chunks = NS * (SPLIT_A + SPLIT_B)
  e_pad = tot_chunks * CHUNK
  assert e_pad >= e, (e_pad, e)
  if e_pad != e:
    pad = e_pad - e
    src = jnp.concatenate([src, jnp.zeros((pad,), jnp.int32)])
    dst = jnp.concatenate([dst, jnp.full((pad,), NPAD - 1, jnp.int32)])
  src2 = src.reshape(tot_chunks, CHUNK)
  dst2 = dst.reshape(tot_chunks, CHUNK)
  extra = MAXSPLIT - SPLIT_B
  if extra:
    src2 = jnp.concatenate([src2, jnp.zeros((extra, CHUNK), jnp.int32)])
    dst2 = jnp.concatenate(
        [dst2, jnp.full((extra, CHUNK), NPAD - 1, jnp.int32)])

  zrow = jnp.zeros((CHUNK, D), jnp.float32)
  zcnt = jnp.zeros((CHUNK, CW), jnp.float32)
  ones = jnp.ones((CHUNK, CW), jnp.float32)

  mesh = plsc.VectorSubcoreMesh(core_axis_name="c", subcore_axis_name="s",
                                num_cores=NC, num_subcores=NS)
  sc_fn = pl.kernel(
      _sc_body,
      out_type=[
          jax.ShapeDtypeStruct((NC, NPAD, D), jnp.float32),
          jax.ShapeDtypeStruct((NC, NPAD, CW), jnp.float32),
      ],
      mesh=mesh,
      compiler_params=pltpu.CompilerParams(use_tc_tiling_on_sc=False),
      scratch_types=[
          pltpu.VMEM((MAXSPLIT, CHUNK), jnp.int32),
          pltpu.VMEM((MAXSPLIT, CHUNK), jnp.int32),
          pltpu.VMEM((CHUNK, D), jnp.float32),
          pltpu.VMEM((CHUNK, D), jnp.float32),
          pltpu.VMEM((CHUNK, CW), jnp.float32),
          pltpu.VMEM((CHUNK, CW), jnp.float32),
          pltpu.VMEM_SHARED((NPAD, D), jnp.float32),
          pltpu.VMEM_SHARED((NPAD, CW), jnp.float32),
          pltpu.SemaphoreType.DMA,
          pltpu.SemaphoreType.DMA,
          pltpu.SemaphoreType.DMA,
          pltpu.SemaphoreType.DMA,
          pltpu.SemaphoreType.DMA,
          pltpu.SemaphoreType.DMA,
      ],
  )
  sums, cnts = sc_fn(node_feats, src2, dst2, zrow, zcnt, ones)

  BR = 400
  assert n % BR == 0
  out = pl.pallas_call(
      _tc_body,
      grid=(n // BR,),
      in_specs=[
          pl.BlockSpec((NC, BR, D), lambda i: (0, i, 0)),
          pl.BlockSpec((NC, BR, CW), lambda i: (0, i, 0)),
          pl.BlockSpec((D, D), lambda i: (0, 0)),
          pl.BlockSpec((1, D), lambda i: (0, 0)),
      ],
      out_specs=pl.BlockSpec((BR, D), lambda i: (i, 0)),
      out_shape=jax.ShapeDtypeStruct((n, D), jnp.float32),
  )(sums, cnts, W, b.reshape(1, D))
  return out


# speedup vs baseline: 1.0606x; 1.0606x over previous
import jax
import jax.numpy as jnp
from jax import lax
from jax.experimental import pallas as pl
from jax.experimental.pallas import tpu as pltpu
from jax.experimental.pallas import tpu_sc as plsc

D = 128

NC = 2
NS = 16
NW = NC * NS

CHUNK = 64
NPAD = 10240
ROWS_PER_TILE = NPAD // NS
CW = 8

SPLIT_A = 200
SPLIT_B = 114
MAXSPLIT = max(SPLIT_A, SPLIT_B)


def _sc_body(feats_hbm, src_hbm, dst_hbm, zrow_hbm, zcnt_hbm, ones_hbm,
             sums_out, cnts_out,
             sidx_v, didx_v, rows0_v, rows1_v, ones_v, cstage_v,
             acc_sh, cnt_sh,
             sem_g0, sem_g1, sem_s0, sem_s1, sem_c0, sem_c1):
  c = lax.axis_index("c")
  s = lax.axis_index("s")

  start = lax.select(c == 0, s * SPLIT_A, NS * SPLIT_A + s * SPLIT_B)
  n_half = lax.select(c == 0, SPLIT_A // 2, SPLIT_B // 2)

  row0 = s * ROWS_PER_TILE

  nz = ROWS_PER_TILE // CHUNK
  pltpu.sync_copy(zrow_hbm, rows0_v)
  pltpu.sync_copy(zcnt_hbm, cstage_v)
  for k in range(nz):
    pltpu.async_copy(rows0_v, acc_sh.at[pl.ds(row0 + k * CHUNK, CHUNK)],
                     sem_s0)
    pltpu.async_copy(cstage_v, cnt_sh.at[pl.ds(row0 + k * CHUNK, CHUNK)],
                     sem_s1)
  pltpu.async_copy(src_hbm.at[pl.ds(start, MAXSPLIT)], sidx_v, sem_g0)
  pltpu.async_copy(dst_hbm.at[pl.ds(start, MAXSPLIT)], didx_v, sem_g1)
  pltpu.async_copy(ones_hbm, ones_v, sem_c0)
  for k in range(nz):
    pltpu.make_async_copy(
        rows0_v, acc_sh.at[pl.ds(row0 + k * CHUNK, CHUNK)], sem_s0).wait()
    pltpu.make_async_copy(
        cstage_v, cnt_sh.at[pl.ds(row0 + k * CHUNK, CHUNK)], sem_s1).wait()
  pltpu.make_async_copy(
      src_hbm.at[pl.ds(start, MAXSPLIT)], sidx_v, sem_g0).wait()
  pltpu.make_async_copy(
      dst_hbm.at[pl.ds(start, MAXSPLIT)], didx_v, sem_g1).wait()
  pltpu.make_async_copy(ones_hbm, ones_v, sem_c0).wait()
  plsc.subcore_barrier()

  H = CHUNK // 2

  def gather(j, rows_v, sem):
    pltpu.async_copy(feats_hbm.at[sidx_v.at[j, pl.ds(0, H)]],
                     rows_v.at[pl.ds(0, H)], sem)
    pltpu.async_copy(feats_hbm.at[sidx_v.at[j, pl.ds(H, H)]],
                     rows_v.at[pl.ds(H, H)], sem)

  def gather_wait(j, rows_v, sem):
    pltpu.make_async_copy(feats_hbm.at[sidx_v.at[j, pl.ds(0, H)]],
                          rows_v.at[pl.ds(0, H)], sem).wait()
    pltpu.make_async_copy(feats_hbm.at[sidx_v.at[j, pl.ds(H, H)]],
                          rows_v.at[pl.ds(H, H)], sem).wait()

  def scatter(j, rows_v, sem):
    return pltpu.async_copy(rows_v, acc_sh.at[didx_v.at[j]], sem, add=True)

  def counts(j, sem):
    return pltpu.async_copy(ones_v, cnt_sh.at[didx_v.at[j]], sem, add=True)

  gather(0, rows0_v, sem_g0)

  def body(i, carry):
    j0 = 2 * i
    j1 = j0 + 1
    gather_wait(j0, rows0_v, sem_g0)
    scatter(j0, rows0_v, sem_s0)

    @pl.when(i > 0)
    def _():
      pltpu.make_async_copy(rows1_v, acc_sh.at[didx_v.at[j1]], sem_s1).wait()
      pltpu.make_async_copy(ones_v, cnt_sh.at[didx_v.at[j1]], sem_c1).wait()

    counts(j0, sem_c0)
    gather(j1, rows1_v, sem_g1)

    gather_wait(j1, rows1_v, sem_g1)
    scatter(j1, rows1_v, sem_s1)
    pltpu.make_async_copy(rows0_v, acc_sh.at[didx_v.at[j0]], sem_s0).wait()
    pltpu.make_async_copy(ones_v, cnt_sh.at[didx_v.at[j0]], sem_c0).wait()
    counts(j1, sem_c1)

    @pl.when(i < n_half - 1)
    def _():
      gather(j0 + 2, rows0_v, sem_g0)

    return carry

  lax.fori_loop(0, n_half, body, 0)
  pltpu.make_async_copy(rows1_v, acc_sh.at[didx_v.at[0]], sem_s1).wait()
  pltpu.make_async_copy(ones_v, cnt_sh.at[didx_v.at[0]], sem_c1).wait()
  plsc.subcore_barrier()

  for k in range(nz):
    pltpu.async_copy(acc_sh.at[pl.ds(row0 + k * CHUNK, CHUNK)],
                     sums_out.at[c, pl.ds(row0 + k * CHUNK, CHUNK)], sem_s0)
    pltpu.async_copy(cnt_sh.at[pl.ds(row0 + k * CHUNK, CHUNK)],
                     cnts_out.at[c, pl.ds(row0 + k * CHUNK, CHUNK)], sem_s1)
  for k in range(nz):
    pltpu.make_async_copy(
        acc_sh.at[pl.ds(row0 + k * CHUNK, CHUNK)],
        sums_out.at[c, pl.ds(row0 + k * CHUNK, CHUNK)], sem_s0).wait()
    pltpu.make_async_copy(
        cnt_sh.at[pl.ds(row0 + k * CHUNK, CHUNK)],
        cnts_out.at[c, pl.ds(row0 + k * CHUNK, CHUNK)], sem_s1).wait()


def _tc_body(sums_ref, cnts_ref, w_ref, b_ref, out_ref):
  ssum = sums_ref[0] + sums_ref[1]
  cnt = cnts_ref[0, :, 0:1] + cnts_ref[1, :, 0:1]
  neigh = ssum / jnp.maximum(cnt, 1.0)
  acc = lax.dot_general(neigh, w_ref[...], (((1,), (1,)), ((), ())),
                        preferred_element_type=jnp.float32)
  out_ref[...] = jnp.maximum(acc + b_ref[...], 0.0)


def kernel(node_feats, edge_index, W, b):
  n, d = node_feats.shape
  e = edge_index.shape[1]
  src = edge_index[0].astype(jnp.int32)
  dst = edge_index[1].astype(jnp.int32)

  tot_chunks = NS * (SPLIT_A + SPLIT_B)
  e_pad = tot_chunks * CHUNK
  assert e_pad >= e, (e_pad, e)
  if e_pad != e:
    pad = e_pad - e
    src = jnp.concatenate([src, jnp.zeros((pad,), jnp.int32)])
    dst = jnp.concatenate([dst, jnp.full((pad,), NPAD - 1, jnp.int32)])
  src2 = src.reshape(tot_chunks, CHUNK)
  dst2 = dst.reshape(tot_chunks, CHUNK)
  extra = MAXSPLIT - SPLIT_B
  if extra:
    src2 = jnp.concatenate([src2, jnp.zeros((extra, CHUNK), jnp.int32)])
    dst2 = jnp.concatenate(
        [dst2, jnp.full((extra, CHUNK), NPAD - 1, jnp.int32)])

  zrow = jnp.zeros((CHUNK, D), jnp.float32)
  zcnt = jnp.zeros((CHUNK, CW), jnp.float32)
  ones = jnp.ones((CHUNK, CW), jnp.float32)

  mesh = plsc.VectorSubcoreMesh(core_axis_name="c", subcore_axis_name="s",
                                num_cores=NC, num_subcores=NS)
  sc_fn = pl.kernel(
      _sc_body,
      out_type=[
          jax.ShapeDtypeStruct((NC, NPAD, D), jnp.float32),
          jax.ShapeDtypeStruct((NC, NPAD, CW), jnp.float32),
      ],
      mesh=mesh,
      compiler_params=pltpu.CompilerParams(use_tc_tiling_on_sc=False),
      scratch_types=[
          pltpu.VMEM((MAXSPLIT, CHUNK), jnp.int32),
          pltpu.VMEM((MAXSPLIT, CHUNK), jnp.int32),
          pltpu.VMEM((CHUNK, D), jnp.float32),
          pltpu.VMEM((CHUNK, D), jnp.float32),
          pltpu.VMEM((CHUNK, CW), jnp.float32),
          pltpu.VMEM((CHUNK, CW), jnp.float32),
          pltpu.VMEM_SHARED((NPAD, D), jnp.float32),
          pltpu.VMEM_SHARED((NPAD, CW), jnp.float32),
          pltpu.SemaphoreType.DMA,
          pltpu.SemaphoreType.DMA,
          pltpu.SemaphoreType.DMA,
          pltpu.SemaphoreType.DMA,
          pltpu.SemaphoreType.DMA,
          pltpu.SemaphoreType.DMA,
      ],
  )
  sums, cnts = sc_fn(node_feats, src2, dst2, zrow, zcnt, ones)

  BR = 400
  assert n % BR == 0
  out = pl.pallas_call(
      _tc_body,
      grid=(n // BR,),
      in_specs=[
          pl.BlockSpec((NC, BR, D), lambda i: (0, i, 0)),
          pl.BlockSpec((NC, BR, CW), lambda i: (0, i, 0)),
          pl.BlockSpec((D, D), lambda i: (0, 0)),
          pl.BlockSpec((1, D), lambda i: (0, 0)),
      ],
      out_specs=pl.BlockSpec((BR, D), lambda i: (i, 0)),
      out_shape=jax.ShapeDtypeStruct((n, D), jnp.float32),
  )(sums, cnts, W, b.reshape(1, D))
  return out


# speedup vs baseline: 1.0611x; 1.0005x over previous
import jax
import jax.numpy as jnp
from jax import lax
from jax.experimental import pallas as pl
from jax.experimental.pallas import tpu as pltpu
from jax.experimental.pallas import tpu_sc as plsc

D = 128

NC = 2
NS = 16
NW = NC * NS

CHUNK = 64
NPAD = 10240
ROWS_PER_TILE = NPAD // NS
CW = 8

SPLIT_A = 200
SPLIT_B = 114
MAXSPLIT = max(SPLIT_A, SPLIT_B)


def _sc_body(feats_hbm, src_hbm, dst_hbm, zrow_hbm, zcnt_hbm, ones_hbm,
             sums_out, cnts_out,
             sidx_v, didx_v, rows0_v, rows1_v, ones_v, cstage_v,
             acc_sh, cnt_sh,
             sem_g0, sem_g1, sem_s0, sem_s1, sem_c0, sem_c1):
  c = lax.axis_index("c")
  s = lax.axis_index("s")

  start = lax.select(c == 0, s * SPLIT_A, NS * SPLIT_A + s * SPLIT_B)
  n_half = lax.select(c == 0, SPLIT_A // 2, SPLIT_B // 2)

  row0 = s * ROWS_PER_TILE

  nz = ROWS_PER_TILE // CHUNK
  pltpu.sync_copy(zrow_hbm, rows0_v)
  pltpu.sync_copy(zcnt_hbm, cstage_v)
  for k in range(nz):
    pltpu.async_copy(rows0_v, acc_sh.at[pl.ds(row0 + k * CHUNK, CHUNK)],
                     sem_s0)
    pltpu.async_copy(cstage_v, cnt_sh.at[pl.ds(row0 + k * CHUNK, CHUNK)],
                     sem_s1)
  pltpu.async_copy(src_hbm.at[pl.ds(start, MAXSPLIT)], sidx_v, sem_g0)
  pltpu.async_copy(dst_hbm.at[pl.ds(start, MAXSPLIT)], didx_v, sem_g1)
  pltpu.async_copy(ones_hbm, ones_v, sem_c0)
  for k in range(nz):
    pltpu.make_async_copy(
        rows0_v, acc_sh.at[pl.ds(row0 + k * CHUNK, CHUNK)], sem_s0).wait()
    pltpu.make_async_copy(
        cstage_v, cnt_sh.at[pl.ds(row0 + k * CHUNK, CHUNK)], sem_s1).wait()
  pltpu.make_async_copy(
      src_hbm.at[pl.ds(start, MAXSPLIT)], sidx_v, sem_g0).wait()
  pltpu.make_async_copy(
      dst_hbm.at[pl.ds(start, MAXSPLIT)], didx_v, sem_g1).wait()
  pltpu.make_async_copy(ones_hbm, ones_v, sem_c0).wait()
  plsc.subcore_barrier()

  H = CHUNK // 2

  def gather(j, rows_v, sem):
    pltpu.async_copy(feats_hbm.at[sidx_v.at[j, pl.ds(0, H)]],
                     rows_v.at[pl.ds(0, H)], sem)
    pltpu.async_copy(feats_hbm.at[sidx_v.at[j, pl.ds(H, H)]],
                     rows_v.at[pl.ds(H, H)], sem)

  def gather_wait(j, rows_v, sem):
    pltpu.make_async_copy(feats_hbm.at[sidx_v.at[j, pl.ds(0, H)]],
                          rows_v.at[pl.ds(0, H)], sem).wait()
    pltpu.make_async_copy(feats_hbm.at[sidx_v.at[j, pl.ds(H, H)]],
                          rows_v.at[pl.ds(H, H)], sem).wait()

  def scatter(j, rows_v, sem):
    return pltpu.async_copy(rows_v, acc_sh.at[didx_v.at[j]], sem, add=True)

  def counts(j, sem):
    return pltpu.async_copy(ones_v, cnt_sh.at[didx_v.at[j]], sem, add=True)

  gather(0, rows0_v, sem_g0)

  def body(i, carry):
    j0 = 2 * i
    j1 = j0 + 1
    gather_wait(j0, rows0_v, sem_g0)
    scatter(j0, rows0_v, sem_s0)
    counts(j0, sem_c0)
    gather(j1, rows1_v, sem_g1)

    gather_wait(j1, rows1_v, sem_g1)
    scatter(j1, rows1_v, sem_s1)
    counts(j1, sem_c1)

    pltpu.make_async_copy(rows0_v, acc_sh.at[didx_v.at[j0]], sem_s0).wait()
    pltpu.make_async_copy(ones_v, cnt_sh.at[didx_v.at[j0]], sem_c0).wait()

    @pl.when(i < n_half - 1)
    def _():
      gather(j0 + 2, rows0_v, sem_g0)

    pltpu.make_async_copy(rows1_v, acc_sh.at[didx_v.at[j1]], sem_s1).wait()
    pltpu.make_async_copy(ones_v, cnt_sh.at[didx_v.at[j1]], sem_c1).wait()
    return carry

  lax.fori_loop(0, n_half, body, 0)
  plsc.subcore_barrier()

  for k in range(nz):
    pltpu.async_copy(acc_sh.at[pl.ds(row0 + k * CHUNK, CHUNK)],
                     sums_out.at[c, pl.ds(row0 + k * CHUNK, CHUNK)], sem_s0)
    pltpu.async_copy(cnt_sh.at[pl.ds(row0 + k * CHUNK, CHUNK)],
                     cnts_out.at[c, pl.ds(row0 + k * CHUNK, CHUNK)], sem_s1)
  for k in range(nz):
    pltpu.make_async_copy(
        acc_sh.at[pl.ds(row0 + k * CHUNK, CHUNK)],
        sums_out.at[c, pl.ds(row0 + k * CHUNK, CHUNK)], sem_s0).wait()
    pltpu.make_async_copy(
        cnt_sh.at[pl.ds(row0 + k * CHUNK, CHUNK)],
        cnts_out.at[c, pl.ds(row0 + k * CHUNK, CHUNK)], sem_s1).wait()


def _tc_body(sums_ref, cnts_ref, w_ref, b_ref, out_ref):
  ssum = sums_ref[0] + sums_ref[1]
  cnt = cnts_ref[0, :, 0:1] + cnts_ref[1, :, 0:1]
  neigh = ssum / jnp.maximum(cnt, 1.0)
  acc = lax.dot_general(neigh, w_ref[...], (((1,), (1,)), ((), ())),
                        preferred_element_type=jnp.float32)
  out_ref[...] = jnp.maximum(acc + b_ref[...], 0.0)


def kernel(node_feats, edge_index, W, b):
  n, d = node_feats.shape
  e = edge_index.shape[1]
  src = edge_index[0].astype(jnp.int32)
  dst = edge_index[1].astype(jnp.int32)

  tot_chunks = NS * (SPLIT_A + SPLIT_B)
  e_pad = tot_chunks * CHUNK
  assert e_pad >= e, (e_pad, e)
  if e_pad != e:
    pad = e_pad - e
    src = jnp.concatenate([src, jnp.zeros((pad,), jnp.int32)])
    dst = jnp.concatenate([dst, jnp.full((pad,), NPAD - 1, jnp.int32)])
  src2 = src.reshape(tot_chunks, CHUNK)
  dst2 = dst.reshape(tot_chunks, CHUNK)
  extra = MAXSPLIT - SPLIT_B
  if extra:
    src2 = jnp.concatenate([src2, jnp.zeros((extra, CHUNK), jnp.int32)])
    dst2 = jnp.concatenate(
        [dst2, jnp.full((extra, CHUNK), NPAD - 1, jnp.int32)])

  zrow = jnp.zeros((CHUNK, D), jnp.float32)
  zcnt = jnp.zeros((CHUNK, CW), jnp.float32)
  ones = jnp.ones((CHUNK, CW), jnp.float32)

  mesh = plsc.VectorSubcoreMesh(core_axis_name="c", subcore_axis_name="s",
                                num_cores=NC, num_subcores=NS)
  sc_fn = pl.kernel(
      _sc_body,
      out_type=[
          jax.ShapeDtypeStruct((NC, NPAD, D), jnp.float32),
          jax.ShapeDtypeStruct((NC, NPAD, CW), jnp.float32),
      ],
      mesh=mesh,
      compiler_params=pltpu.CompilerParams(use_tc_tiling_on_sc=False),
      scratch_types=[
          pltpu.VMEM((MAXSPLIT, CHUNK), jnp.int32),
          pltpu.VMEM((MAXSPLIT, CHUNK), jnp.int32),
          pltpu.VMEM((CHUNK, D), jnp.float32),
          pltpu.VMEM((CHUNK, D), jnp.float32),
          pltpu.VMEM((CHUNK, CW), jnp.float32),
          pltpu.VMEM((CHUNK, CW), jnp.float32),
          pltpu.VMEM_SHARED((NPAD, D), jnp.float32),
          pltpu.VMEM_SHARED((NPAD, CW), jnp.float32),
          pltpu.SemaphoreType.DMA,
          pltpu.SemaphoreType.DMA,
          pltpu.SemaphoreType.DMA,
          pltpu.SemaphoreType.DMA,
          pltpu.SemaphoreType.DMA,
          pltpu.SemaphoreType.DMA,
      ],
  )
  sums, cnts = sc_fn(node_feats, src2, dst2, zrow, zcnt, ones)

  BR = 400
  assert n % BR == 0
  out = pl.pallas_call(
      _tc_body,
      grid=(n // BR,),
      in_specs=[
          pl.BlockSpec((NC, BR, D), lambda i: (0, i, 0)),
          pl.BlockSpec((NC, BR, CW), lambda i: (0, i, 0)),
          pl.BlockSpec((D, D), lambda i: (0, 0)),
          pl.BlockSpec((1, D), lambda i: (0, 0)),
      ],
      out_specs=pl.BlockSpec((BR, D), lambda i: (i, 0)),
      out_shape=jax.ShapeDtypeStruct((n, D), jnp.float32),
  )(sums, cnts, W, b.reshape(1, D))
  return out
